# baseline (speedup 1.0000x reference)
import functools

import jax
import jax.numpy as jnp
from jax import lax
from jax.experimental import pallas as pl
from jax.experimental.pallas import tpu as pltpu
from jax.experimental.pallas import tpu_sc as plsc

N_NODES = 100000
MEM_DIM = 128
MSG_DIM = 256
BATCH = 16384

NC = 2
NS = 16
NW = NC * NS
LANES = 16

CHUNK = BATCH // NW
NVEC = BATCH // LANES
HVEC = NVEC // NC
RANGE = (N_NODES + NS - 1) // NS

_mesh = plsc.VectorSubcoreMesh(
    core_axis_name="c", subcore_axis_name="s", num_cores=NC, num_subcores=NS
)
_sc_params = pltpu.CompilerParams(needs_layout_passes=False)


def _wid():
    return lax.axis_index("s") * NC + lax.axis_index("c")


@functools.partial(
    pl.kernel,
    out_type=jax.ShapeDtypeStruct((BATCH, MEM_DIM), jnp.float32),
    mesh=_mesh,
    compiler_params=_sc_params,
    scratch_types=[
        pltpu.VMEM((CHUNK,), jnp.int32),
        pltpu.VMEM((CHUNK, MEM_DIM), jnp.float32),
        pltpu.SemaphoreType.DMA,
    ],
)
def _gather(mem_hbm, ids_hbm, h_hbm, ids_l, rows, sem):
    base = _wid() * CHUNK
    pltpu.sync_copy(ids_hbm.at[pl.ds(base, CHUNK)], ids_l)
    cps = [
        pltpu.async_copy(mem_hbm.at[ids_l.at[pl.ds(j * 128, 128)]],
                         rows.at[pl.ds(j * 128, 128)], sem)
        for j in range(4)
    ]
    for cp in cps:
        cp.wait()
    pltpu.sync_copy(rows, h_hbm.at[pl.ds(base, CHUNK)])


@functools.partial(
    pl.kernel,
    out_type=jax.ShapeDtypeStruct((NS, BATCH), jnp.int32),
    mesh=_mesh,
    compiler_params=_sc_params,
    scratch_types=[
        pltpu.VMEM((BATCH,), jnp.int32),
        pltpu.VMEM((RANGE + 16,), jnp.int32),
        pltpu.VMEM((BATCH // NC,), jnp.int32),
    ],
)
def _dedup(ids_hbm, adj_hbm, ids_v, winner, adj_buf):
    s = lax.axis_index("s")
    k = lax.axis_index("c")
    lo = s * RANGE
    hi = lo + RANGE
    pltpu.sync_copy(ids_hbm, ids_v)

    lane = lax.iota(jnp.int32, LANES)

    def p1(i, _):
        v = ids_v[pl.ds(i * LANES, LANES)]
        m = (v >= lo) & (v < hi)
        plsc.store_scatter(winner, [v - lo], lane + i * LANES, mask=m)
        return 0

    lax.fori_loop(0, NVEC, p1, 0, unroll=8)

    vbase = k * HVEC

    def p2(j, _):
        i = vbase + j
        v = ids_v[pl.ds(i * LANES, LANES)]
        m = (v >= lo) & (v < hi)
        w = plsc.load_gather(winner, [v - lo], mask=m)
        adj_buf[pl.ds(j * LANES, LANES)] = jnp.where(m, w, 0)
        return 0

    lax.fori_loop(0, HVEC, p2, 0, unroll=8)
    pltpu.sync_copy(adj_buf, adj_hbm.at[s, pl.ds(k * (BATCH // NC), BATCH // NC)])


@functools.partial(
    pl.kernel,
    out_type=(),
    mesh=_mesh,
    compiler_params=_sc_params,
    scratch_types=[
        pltpu.VMEM((8, 128), jnp.int32),
        pltpu.VMEM((4, 128), jnp.int32),
        pltpu.VMEM((NS, CHUNK), jnp.int32),
        pltpu.VMEM((BATCH,), jnp.float32),
        pltpu.VMEM((CHUNK, MEM_DIM), jnp.float32),
        pltpu.VMEM((4, 128), jnp.float32),
        pltpu.SemaphoreType.DMA,
        pltpu.SemaphoreType.DMA,
        pltpu.SemaphoreType.DMA((4,)),
        pltpu.SemaphoreType.DMA((4,)),
    ],
)
def _scatter(rows_hbm, ids128_hbm, ts_hbm, adj2d_hbm, mem_ref, lu_ref,
             idx8, adj2, a16, ts_v, rows, ts4, sem_i, sem_a, semg, semt):
    c = _wid()
    base = c * CHUNK
    roff = 4 * (c % 2)
    cpi = pltpu.async_copy(ids128_hbm.at[pl.ds(8 * (c // 2), 8)], idx8, sem_i)
    cpt = pltpu.async_copy(ts_hbm, ts_v, sem_i)
    cpa = pltpu.async_copy(adj2d_hbm.at[:, pl.ds(base, CHUNK)], a16, sem_a)
    cpa.wait()
    for p in range(NW):
        acc = a16[0, pl.ds(p * LANES, LANES)]
        for j in range(1, NS):
            acc = acc + a16[j, pl.ds(p * LANES, LANES)]
        adj2[p // 8, pl.ds((p % 8) * LANES, LANES)] = acc
    cpi.wait()
    g = [
        pltpu.async_copy(rows_hbm.at[adj2.at[j]], rows.at[pl.ds(j * 128, 128)],
                         semg.at[j])
        for j in range(4)
    ]
    cpt.wait()
    for p in range(NW):
        av = adj2[p // 8, pl.ds((p % 8) * LANES, LANES)]
        ts4[p // 8, pl.ds((p % 8) * LANES, LANES)] = plsc.load_gather(ts_v, [av])
    s, s2 = [], []
    for j in range(4):
        g[j].wait()
        s.append(pltpu.async_copy(rows.at[pl.ds(j * 128, 128)],
                                  mem_ref.at[idx8.at[roff + j]], semg.at[j]))
        s2.append(pltpu.async_copy(ts4.at[j], lu_ref.at[idx8.at[roff + j]],
                                   semt.at[j]))
    for cp in s + s2:
        cp.wait()


GRU_BLK = 512
MEM_BLK = 3128


def _gru_body(x_ref, h_ref, wih_ref, whh_ref, bih_ref, bhh_ref, memin_ref,
              out_ref, memout_ref):
    x = x_ref[...]
    h = h_ref[...]
    dn = (((1,), (1,)), ((), ()))
    gi = lax.dot_general(x, wih_ref[...], dn, preferred_element_type=jnp.float32)
    gi = gi + bih_ref[...]
    gh = lax.dot_general(h, whh_ref[...], dn, preferred_element_type=jnp.float32)
    gh = gh + bhh_ref[...]
    H = MEM_DIM
    r = jax.nn.sigmoid(gi[:, :H] + gh[:, :H])
    z = jax.nn.sigmoid(gi[:, H:2 * H] + gh[:, H:2 * H])
    n = jnp.tanh(gi[:, 2 * H:] + r * gh[:, 2 * H:])
    out_ref[...] = (1.0 - z) * n + z * h
    memout_ref[...] = memin_ref[...]


_gru = pl.pallas_call(
    _gru_body,
    grid=(BATCH // GRU_BLK,),
    in_specs=[
        pl.BlockSpec((GRU_BLK, MSG_DIM), lambda i: (i, 0)),
        pl.BlockSpec((GRU_BLK, MEM_DIM), lambda i: (i, 0)),
        pl.BlockSpec((3 * MEM_DIM, MSG_DIM), lambda i: (0, 0)),
        pl.BlockSpec((3 * MEM_DIM, MEM_DIM), lambda i: (0, 0)),
        pl.BlockSpec((1, 3 * MEM_DIM), lambda i: (0, 0)),
        pl.BlockSpec((1, 3 * MEM_DIM), lambda i: (0, 0)),
        pl.BlockSpec((MEM_BLK, MEM_DIM), lambda i: (i, 0)),
    ],
    out_specs=(
        pl.BlockSpec((GRU_BLK, MEM_DIM), lambda i: (i, 0)),
        pl.BlockSpec((MEM_BLK, MEM_DIM), lambda i: (i, 0)),
    ),
    out_shape=(
        jax.ShapeDtypeStruct((BATCH, MEM_DIM), jnp.float32),
        jax.ShapeDtypeStruct((N_NODES, MEM_DIM), jnp.float32),
    ),
)


def kernel(node_memory, last_update, unique_node_ids, unique_messages,
           timestamps, W_ih, W_hh, b_ih, b_hh):
    ids = unique_node_ids.astype(jnp.int32)
    h = _gather(node_memory, ids)
    adj = _dedup(ids)
    rows, mem_copy = _gru(unique_messages, h, W_ih, W_hh,
                          b_ih.reshape(1, -1), b_hh.reshape(1, -1),
                          node_memory)
    mem_ref = jax.new_ref(mem_copy)
    lu_ref = jax.new_ref(last_update)
    _scatter(rows, ids.reshape(BATCH // 128, 128), timestamps, adj,
             mem_ref, lu_ref)
    return mem_ref[...], lu_ref[...]


# speedup vs baseline: 1.0866x; 1.0866x over previous
import functools

import jax
import jax.numpy as jnp
from jax import lax
from jax.experimental import pallas as pl
from jax.experimental.pallas import tpu as pltpu
from jax.experimental.pallas import tpu_sc as plsc

N_NODES = 100000
MEM_DIM = 128
MSG_DIM = 256
BATCH = 16384

NC = 2
NS = 16
NW = NC * NS
LANES = 16

CHUNK = BATCH // NW
NVEC = BATCH // LANES
HVEC = NVEC // NC
RANGE = (N_NODES + NS - 1) // NS

_mesh = plsc.VectorSubcoreMesh(
    core_axis_name="c", subcore_axis_name="s", num_cores=NC, num_subcores=NS
)
_sc_params = pltpu.CompilerParams(needs_layout_passes=False)


def _wid():
    return lax.axis_index("s") * NC + lax.axis_index("c")


@functools.partial(
    pl.kernel,
    out_type=jax.ShapeDtypeStruct((BATCH, MEM_DIM), jnp.float32),
    mesh=_mesh,
    compiler_params=_sc_params,
    scratch_types=[
        pltpu.VMEM((CHUNK,), jnp.int32),
        pltpu.VMEM((CHUNK, MEM_DIM), jnp.float32),
        pltpu.SemaphoreType.DMA,
    ],
)
def _gather(mem_hbm, ids_hbm, h_hbm, ids_l, rows, sem):
    base = _wid() * CHUNK
    pltpu.sync_copy(ids_hbm.at[pl.ds(base, CHUNK)], ids_l)
    cps = [
        pltpu.async_copy(mem_hbm.at[ids_l.at[pl.ds(j * 128, 128)]],
                         rows.at[pl.ds(j * 128, 128)], sem)
        for j in range(4)
    ]
    for cp in cps:
        cp.wait()
    pltpu.sync_copy(rows, h_hbm.at[pl.ds(base, CHUNK)])


@functools.partial(
    pl.kernel,
    out_type=jax.ShapeDtypeStruct((NS, BATCH), jnp.int32),
    mesh=_mesh,
    compiler_params=_sc_params,
    scratch_types=[
        pltpu.VMEM((BATCH,), jnp.int32),
        pltpu.VMEM((RANGE + 16,), jnp.int32),
        pltpu.VMEM((BATCH // NC,), jnp.int32),
    ],
)
def _dedup(ids_hbm, adj_hbm, ids_v, winner, adj_buf):
    s = lax.axis_index("s")
    k = lax.axis_index("c")
    lo = s * RANGE
    hi = lo + RANGE
    pltpu.sync_copy(ids_hbm, ids_v)

    lane = lax.iota(jnp.int32, LANES)

    def p1(i, _):
        v = ids_v[pl.ds(i * LANES, LANES)]
        m = (v >= lo) & (v < hi)
        plsc.store_scatter(winner, [v - lo], lane + i * LANES, mask=m)
        return 0

    lax.fori_loop(0, NVEC, p1, 0, unroll=8)

    vbase = k * HVEC

    def p2(j, _):
        i = vbase + j
        v = ids_v[pl.ds(i * LANES, LANES)]
        m = (v >= lo) & (v < hi)
        w = plsc.load_gather(winner, [v - lo], mask=m)
        adj_buf[pl.ds(j * LANES, LANES)] = jnp.where(m, w, 0)
        return 0

    lax.fori_loop(0, HVEC, p2, 0, unroll=8)
    pltpu.sync_copy(adj_buf, adj_hbm.at[s, pl.ds(k * (BATCH // NC), BATCH // NC)])


@functools.partial(
    pl.kernel,
    out_type=(),
    mesh=_mesh,
    compiler_params=_sc_params,
    scratch_types=[
        pltpu.VMEM((8, 128), jnp.int32),
        pltpu.VMEM((4, 128), jnp.int32),
        pltpu.VMEM((NS, CHUNK), jnp.int32),
        pltpu.VMEM((BATCH,), jnp.float32),
        pltpu.VMEM((CHUNK, MEM_DIM), jnp.float32),
        pltpu.VMEM((4, 128), jnp.float32),
        pltpu.SemaphoreType.DMA,
        pltpu.SemaphoreType.DMA,
        pltpu.SemaphoreType.DMA((4,)),
        pltpu.SemaphoreType.DMA((4,)),
    ],
)
def _scatter(rows_hbm, ids128_hbm, ts_hbm, adj2d_hbm, mem_ref, lu_ref,
             idx8, adj2, a16, ts_v, rows, ts4, sem_i, sem_a, semg, semt):
    c = _wid()
    base = c * CHUNK
    roff = 4 * (c % 2)
    cpi = pltpu.async_copy(ids128_hbm.at[pl.ds(8 * (c // 2), 8)], idx8, sem_i)
    cpt = pltpu.async_copy(ts_hbm, ts_v, sem_i)
    cpa = pltpu.async_copy(adj2d_hbm.at[:, pl.ds(base, CHUNK)], a16, sem_a)
    cpa.wait()
    for p in range(NW):
        acc = a16[0, pl.ds(p * LANES, LANES)]
        for j in range(1, NS):
            acc = acc + a16[j, pl.ds(p * LANES, LANES)]
        adj2[p // 8, pl.ds((p % 8) * LANES, LANES)] = acc
    cpi.wait()
    g = [
        pltpu.async_copy(rows_hbm.at[adj2.at[j]], rows.at[pl.ds(j * 128, 128)],
                         semg.at[j])
        for j in range(4)
    ]
    cpt.wait()
    for p in range(NW):
        av = adj2[p // 8, pl.ds((p % 8) * LANES, LANES)]
        ts4[p // 8, pl.ds((p % 8) * LANES, LANES)] = plsc.load_gather(ts_v, [av])
    s, s2 = [], []
    for j in range(4):
        g[j].wait()
        s.append(pltpu.async_copy(rows.at[pl.ds(j * 128, 128)],
                                  mem_ref.at[idx8.at[roff + j]], semg.at[j]))
        s2.append(pltpu.async_copy(ts4.at[j], lu_ref.at[idx8.at[roff + j]],
                                   semt.at[j]))
    for cp in s + s2:
        cp.wait()


GRU_BLK = 2048
MEM_BLK = 12512


def _gru_body(x_ref, h_ref, wih_ref, whh_ref, bih_ref, bhh_ref, memin_ref,
              out_ref, memout_ref):
    x = x_ref[...]
    h = h_ref[...]
    dn = (((1,), (1,)), ((), ()))
    gi = lax.dot_general(x, wih_ref[...], dn, preferred_element_type=jnp.float32)
    gi = gi + bih_ref[...]
    gh = lax.dot_general(h, whh_ref[...], dn, preferred_element_type=jnp.float32)
    gh = gh + bhh_ref[...]
    H = MEM_DIM
    r = jax.nn.sigmoid(gi[:, :H] + gh[:, :H])
    z = jax.nn.sigmoid(gi[:, H:2 * H] + gh[:, H:2 * H])
    n = jnp.tanh(gi[:, 2 * H:] + r * gh[:, 2 * H:])
    out_ref[...] = (1.0 - z) * n + z * h
    memout_ref[...] = memin_ref[...]


_gru = pl.pallas_call(
    _gru_body,
    grid=(BATCH // GRU_BLK,),
    in_specs=[
        pl.BlockSpec((GRU_BLK, MSG_DIM), lambda i: (i, 0)),
        pl.BlockSpec((GRU_BLK, MEM_DIM), lambda i: (i, 0)),
        pl.BlockSpec((3 * MEM_DIM, MSG_DIM), lambda i: (0, 0)),
        pl.BlockSpec((3 * MEM_DIM, MEM_DIM), lambda i: (0, 0)),
        pl.BlockSpec((1, 3 * MEM_DIM), lambda i: (0, 0)),
        pl.BlockSpec((1, 3 * MEM_DIM), lambda i: (0, 0)),
        pl.BlockSpec((MEM_BLK, MEM_DIM), lambda i: (i, 0)),
    ],
    out_specs=(
        pl.BlockSpec((GRU_BLK, MEM_DIM), lambda i: (i, 0)),
        pl.BlockSpec((MEM_BLK, MEM_DIM), lambda i: (i, 0)),
    ),
    out_shape=(
        jax.ShapeDtypeStruct((BATCH, MEM_DIM), jnp.float32),
        jax.ShapeDtypeStruct((N_NODES, MEM_DIM), jnp.float32),
    ),
)


def kernel(node_memory, last_update, unique_node_ids, unique_messages,
           timestamps, W_ih, W_hh, b_ih, b_hh):
    ids = unique_node_ids.astype(jnp.int32)
    h = _gather(node_memory, ids)
    adj = _dedup(ids)
    rows, mem_copy = _gru(unique_messages, h, W_ih, W_hh,
                          b_ih.reshape(1, -1), b_hh.reshape(1, -1),
                          node_memory)
    mem_ref = jax.new_ref(mem_copy)
    lu_ref = jax.new_ref(last_update)
    _scatter(rows, ids.reshape(BATCH // 128, 128), timestamps, adj,
             mem_ref, lu_ref)
    return mem_ref[...], lu_ref[...]
